# Initial kernel scaffold; baseline (speedup 1.0000x reference)
import functools

import jax
import jax.numpy as jnp
from jax import lax
from jax.experimental import pallas as pl
from jax.experimental.pallas import tpu as pltpu
from jax.experimental.pallas import tpu_sc as plsc

NV = 1000
NF = 26
D = 64
B = 16384
ROW_STRIDE = NV + 1
FEAT_STRIDE = NF * (NV + 1)


_TC_BLK = 2048


def _idx_body(x_ref, i_ref, s_ref, xt_ref, ipos_ref):
    I = i_ref[...]
    X = x_ref[...]
    S = s_ref[...]
    mx = jnp.max(I, axis=1, keepdims=True)
    j = lax.broadcasted_iota(jnp.int32, I.shape, 1)
    ipos = jnp.min(jnp.where(I == mx, j, NF), axis=1)
    xt = ipos[:, None] * FEAT_STRIDE + j * ROW_STRIDE + X * S + (1 - S) * NV
    xt_ref[...] = xt
    ipos_ref[...] = ipos


def _tc_index(X, I, S):
    return pl.pallas_call(
        _idx_body,
        grid=(B // _TC_BLK,),
        in_specs=[
            pl.BlockSpec((_TC_BLK, NF), lambda i: (i, 0)),
            pl.BlockSpec((_TC_BLK, NF), lambda i: (i, 0)),
            pl.BlockSpec((_TC_BLK, NF), lambda i: (i, 0)),
        ],
        out_specs=[
            pl.BlockSpec((_TC_BLK, NF), lambda i: (i, 0)),
            pl.BlockSpec((_TC_BLK,), lambda i: (i,)),
        ],
        out_shape=[
            jax.ShapeDtypeStruct((B, NF), jnp.int32),
            jax.ShapeDtypeStruct((B,), jnp.int32),
        ],
    )(X, I, S)


_NW = 32
_RPT = B // _NW
_CB = 16
_NCHUNK = _RPT // _CB
_IDX = _CB * NF
_GSPLIT = 4
_GN = _IDX // _GSPLIT


def _sc_body(xt_hbm, ipos_hbm, w_hbm, b_hbm, out_hbm,
             xt_v, ipos_v, rows_v, bias_v, out_v, sem_b, sem_w):
    c = lax.axis_index("c")
    s = lax.axis_index("s")
    wid = s * 2 + c

    def chunk(g, carry):
        base = wid * _RPT + g * _CB
        pltpu.sync_copy(xt_hbm.at[pl.ds(base * NF, _IDX)], xt_v)
        pltpu.sync_copy(ipos_hbm.at[pl.ds(base, _CB)], ipos_v)
        cp_b = pltpu.async_copy(b_hbm.at[ipos_v], bias_v, sem_b)
        cps = []
        for i in range(_GSPLIT):
            cps.append(pltpu.async_copy(
                w_hbm.at[xt_v.at[pl.ds(i * _GN, _GN)]],
                rows_v.at[pl.ds(i * _GN, _GN)], sem_w))
        cp_b.wait()
        for cp in cps:
            cp.wait()

        def row(r, carry2):
            p0 = r * NF
            for k in range(D // 16):
                acc = rows_v[p0, pl.ds(k * 16, 16)]
                for jj in range(1, NF):
                    acc = acc + rows_v[p0 + jj, pl.ds(k * 16, 16)]
                out_v[r, pl.ds(k * 16, 16)] = (
                    acc * (1.0 / NF) + bias_v[r, pl.ds(k * 16, 16)])
            return carry2

        lax.fori_loop(0, _CB, row, 0)
        pltpu.sync_copy(out_v, out_hbm.at[pl.ds(base, _CB), :])
        return carry

    lax.fori_loop(0, _NCHUNK, chunk, 0)


@functools.partial(
    pl.kernel,
    out_type=jax.ShapeDtypeStruct((B, D), jnp.float32),
    mesh=plsc.VectorSubcoreMesh(core_axis_name="c", subcore_axis_name="s"),
    scratch_types=[
        pltpu.VMEM((_IDX,), jnp.int32),
        pltpu.VMEM((_CB,), jnp.int32),
        pltpu.VMEM((_IDX, D), jnp.float32),
        pltpu.VMEM((_CB, D), jnp.float32),
        pltpu.VMEM((_CB, D), jnp.float32),
        pltpu.SemaphoreType.DMA,
        pltpu.SemaphoreType.DMA,
    ],
)
def _sc_gather(xt_hbm, ipos_hbm, w_hbm, b_hbm, out_hbm,
               xt_v, ipos_v, rows_v, bias_v, out_v, sem_b, sem_w):
    _sc_body(xt_hbm, ipos_hbm, w_hbm, b_hbm, out_hbm,
             xt_v, ipos_v, rows_v, bias_v, out_v, sem_b, sem_w)


def kernel(X, I, S, weights, bias):
    X = X.astype(jnp.int32)
    S = S.astype(jnp.int32)
    I = I.astype(jnp.float32)
    xt, ipos = _tc_index(X, I, S)
    return _sc_gather(xt.reshape(-1), ipos, weights, bias)


# speedup vs baseline: 1.1836x; 1.1836x over previous
import functools

import jax
import jax.numpy as jnp
from jax import lax
from jax.experimental import pallas as pl
from jax.experimental.pallas import tpu as pltpu
from jax.experimental.pallas import tpu_sc as plsc

NV = 1000
NF = 26
D = 64
B = 16384
ROW_STRIDE = NV + 1
FEAT_STRIDE = NF * (NV + 1)


_TC_BLK = 2048


def _idx_body(x_ref, i_ref, s_ref, xt_ref, ipos_ref):
    I = i_ref[...]
    X = x_ref[...]
    S = s_ref[...]
    mx = jnp.max(I, axis=1, keepdims=True)
    j = lax.broadcasted_iota(jnp.int32, I.shape, 1)
    ipos = jnp.min(jnp.where(I == mx, j, NF), axis=1)
    xt = ipos[:, None] * FEAT_STRIDE + j * ROW_STRIDE + X * S + (1 - S) * NV
    xt_ref[...] = xt
    ipos_ref[...] = ipos


def _tc_index(X, I, S):
    return pl.pallas_call(
        _idx_body,
        grid=(B // _TC_BLK,),
        in_specs=[
            pl.BlockSpec((_TC_BLK, NF), lambda i: (i, 0)),
            pl.BlockSpec((_TC_BLK, NF), lambda i: (i, 0)),
            pl.BlockSpec((_TC_BLK, NF), lambda i: (i, 0)),
        ],
        out_specs=[
            pl.BlockSpec((_TC_BLK, NF), lambda i: (i, 0)),
            pl.BlockSpec((_TC_BLK,), lambda i: (i,)),
        ],
        out_shape=[
            jax.ShapeDtypeStruct((B, NF), jnp.int32),
            jax.ShapeDtypeStruct((B,), jnp.int32),
        ],
    )(X, I, S)


_NW = 32
_RPT = B // _NW
_CB = 16
_NCHUNK = _RPT // _CB
_IDX = _CB * NF
_GSPLIT = 4
_GN = _IDX // _GSPLIT


def _sc_body(xt_hbm, ipos_hbm, w_hbm, b_hbm, out_hbm,
             xt_v, ipos_v, rows_v, bias_v, out_v, sem_b, sem_w):
    c = lax.axis_index("c")
    s = lax.axis_index("s")
    wid = s * 2 + c

    def chunk(g, carry):
        base = wid * _RPT + g * _CB
        pltpu.sync_copy(xt_hbm.at[pl.ds(base * NF, _IDX)], xt_v)
        pltpu.sync_copy(ipos_hbm.at[pl.ds(base, _CB)], ipos_v)
        cp_b = pltpu.async_copy(b_hbm.at[ipos_v], bias_v, sem_b)
        cps = []
        for i in range(_GSPLIT):
            cps.append(pltpu.async_copy(
                w_hbm.at[xt_v.at[pl.ds(i * _GN, _GN)]],
                rows_v.at[pl.ds(i * _GN, _GN)], sem_w))
        cp_b.wait()
        for cp in cps:
            cp.wait()

        def row(r, carry2):
            p0 = r * NF
            for k in range(D // 16):
                acc = rows_v[p0, pl.ds(k * 16, 16)]
                for jj in range(1, NF):
                    acc = acc + rows_v[p0 + jj, pl.ds(k * 16, 16)]
                out_v[r, pl.ds(k * 16, 16)] = (
                    acc * (1.0 / NF) + bias_v[r, pl.ds(k * 16, 16)])
            return carry2

        lax.fori_loop(0, _CB, row, 0)
        pltpu.sync_copy(out_v, out_hbm.at[pl.ds(base, _CB), :])
        return carry

    lax.fori_loop(0, _NCHUNK, chunk, 0)


@functools.partial(
    pl.kernel,
    out_type=jax.ShapeDtypeStruct((B, D), jnp.float32),
    mesh=plsc.VectorSubcoreMesh(core_axis_name="c", subcore_axis_name="s"),
    compiler_params=pltpu.CompilerParams(use_tc_tiling_on_sc=False),
    scratch_types=[
        pltpu.VMEM((_IDX,), jnp.int32),
        pltpu.VMEM((_CB,), jnp.int32),
        pltpu.VMEM((_IDX, D), jnp.float32),
        pltpu.VMEM((_CB, D), jnp.float32),
        pltpu.VMEM((_CB, D), jnp.float32),
        pltpu.SemaphoreType.DMA,
        pltpu.SemaphoreType.DMA,
    ],
)
def _sc_gather(xt_hbm, ipos_hbm, w_hbm, b_hbm, out_hbm,
               xt_v, ipos_v, rows_v, bias_v, out_v, sem_b, sem_w):
    _sc_body(xt_hbm, ipos_hbm, w_hbm, b_hbm, out_hbm,
             xt_v, ipos_v, rows_v, bias_v, out_v, sem_b, sem_w)


def kernel(X, I, S, weights, bias):
    X = X.astype(jnp.int32)
    S = S.astype(jnp.int32)
    I = I.astype(jnp.float32)
    xt, ipos = _tc_index(X, I, S)
    return _sc_gather(xt.reshape(-1), ipos, weights, bias)


# speedup vs baseline: 1.3406x; 1.1326x over previous
import functools

import jax
import jax.numpy as jnp
from jax import lax
from jax.experimental import pallas as pl
from jax.experimental.pallas import tpu as pltpu
from jax.experimental.pallas import tpu_sc as plsc

NV = 1000
NF = 26
D = 64
B = 16384
ROW_STRIDE = NV + 1
FEAT_STRIDE = NF * (NV + 1)


_TC_BLK = 2048


def _idx_body(x_ref, i_ref, s_ref, xt_ref, ipos_ref):
    I = i_ref[...]
    X = x_ref[...]
    S = s_ref[...]
    mx = jnp.max(I, axis=1, keepdims=True)
    j = lax.broadcasted_iota(jnp.int32, I.shape, 1)
    ipos = jnp.min(jnp.where(I == mx, j, NF), axis=1)
    xt = ipos[:, None] * FEAT_STRIDE + j * ROW_STRIDE + X * S + (1 - S) * NV
    xt_ref[...] = xt
    ipos_ref[...] = ipos


def _tc_index(X, I, S):
    return pl.pallas_call(
        _idx_body,
        grid=(B // _TC_BLK,),
        in_specs=[
            pl.BlockSpec((_TC_BLK, NF), lambda i: (i, 0)),
            pl.BlockSpec((_TC_BLK, NF), lambda i: (i, 0)),
            pl.BlockSpec((_TC_BLK, NF), lambda i: (i, 0)),
        ],
        out_specs=[
            pl.BlockSpec((_TC_BLK, NF), lambda i: (i, 0)),
            pl.BlockSpec((_TC_BLK,), lambda i: (i,)),
        ],
        out_shape=[
            jax.ShapeDtypeStruct((B, NF), jnp.int32),
            jax.ShapeDtypeStruct((B,), jnp.int32),
        ],
    )(X, I, S)


_NW = 32
_RPT = B // _NW
_CB = 16
_NCHUNK = _RPT // _CB
_IDX = _CB * NF
_GSPLIT = 4
_GN = _IDX // _GSPLIT


def _sc_body(xt_hbm, ipos_hbm, w_hbm, b_hbm, out_hbm,
             xt_t, ipos_t, rows_v, bias_v, out_v, sem_w, sem_b, sem_o):
    c = lax.axis_index("c")
    s = lax.axis_index("s")
    wid = s * 2 + c
    tbase = wid * _RPT

    pltpu.sync_copy(xt_hbm.at[pl.ds(tbase * NF, _RPT * NF)], xt_t)
    pltpu.sync_copy(ipos_hbm.at[pl.ds(tbase, _RPT)], ipos_t)

    def issue(g, pb):
        for i in range(_GSPLIT):
            pltpu.async_copy(
                w_hbm.at[xt_t.at[pl.ds(g * _IDX + i * _GN, _GN)]],
                rows_v.at[pb].at[pl.ds(i * _GN, _GN)], sem_w[pb])
        pltpu.async_copy(b_hbm.at[ipos_t.at[pl.ds(g * _CB, _CB)]],
                         bias_v.at[pb], sem_b[pb])

    def wait_in(g, pb):
        for i in range(_GSPLIT):
            pltpu.make_async_copy(
                w_hbm.at[xt_t.at[pl.ds(g * _IDX + i * _GN, _GN)]],
                rows_v.at[pb].at[pl.ds(i * _GN, _GN)], sem_w[pb]).wait()
        pltpu.make_async_copy(b_hbm.at[ipos_t.at[pl.ds(g * _CB, _CB)]],
                              bias_v.at[pb], sem_b[pb]).wait()

    def out_desc(g, pb):
        return pltpu.make_async_copy(
            out_v.at[pb], out_hbm.at[pl.ds(tbase + g * _CB, _CB), :],
            sem_o[pb])

    def finish(g, pb):
        wait_in(g, pb)

        @pl.when(g >= 2)
        def _():
            out_desc(g - 2, pb).wait()

        def row(r, carry2):
            p0 = r * NF
            acc = [rows_v[pb, p0, pl.ds(k * 16, 16)] for k in range(D // 16)]
            for jj in range(1, NF):
                for k in range(D // 16):
                    acc[k] = acc[k] + rows_v[pb, p0 + jj, pl.ds(k * 16, 16)]
            for k in range(D // 16):
                out_v[pb, r, pl.ds(k * 16, 16)] = (
                    acc[k] * (1.0 / NF) + bias_v[pb, r, pl.ds(k * 16, 16)])
            return carry2

        lax.fori_loop(0, _CB, row, 0)
        out_desc(g, pb).start()

    issue(0, 0)

    def body(t, carry):
        g0 = 2 * t
        issue(g0 + 1, 1)
        finish(g0, 0)

        @pl.when(g0 + 2 < _NCHUNK)
        def _():
            issue(g0 + 2, 0)

        finish(g0 + 1, 1)
        return carry

    lax.fori_loop(0, _NCHUNK // 2, body, 0)

    out_desc(_NCHUNK - 2, 0).wait()
    out_desc(_NCHUNK - 1, 1).wait()


@functools.partial(
    pl.kernel,
    out_type=jax.ShapeDtypeStruct((B, D), jnp.float32),
    mesh=plsc.VectorSubcoreMesh(core_axis_name="c", subcore_axis_name="s"),
    compiler_params=pltpu.CompilerParams(use_tc_tiling_on_sc=False),
    scratch_types=[
        pltpu.VMEM((_RPT * NF,), jnp.int32),
        pltpu.VMEM((_RPT,), jnp.int32),
        pltpu.VMEM((2, _IDX, D), jnp.float32),
        pltpu.VMEM((2, _CB, D), jnp.float32),
        pltpu.VMEM((2, _CB, D), jnp.float32),
        (pltpu.SemaphoreType.DMA, pltpu.SemaphoreType.DMA),
        (pltpu.SemaphoreType.DMA, pltpu.SemaphoreType.DMA),
        (pltpu.SemaphoreType.DMA, pltpu.SemaphoreType.DMA),
    ],
)
def _sc_gather(xt_hbm, ipos_hbm, w_hbm, b_hbm, out_hbm,
               xt_t, ipos_t, rows_v, bias_v, out_v, sem_w, sem_b, sem_o):
    _sc_body(xt_hbm, ipos_hbm, w_hbm, b_hbm, out_hbm,
             xt_t, ipos_t, rows_v, bias_v, out_v, sem_w, sem_b, sem_o)


def kernel(X, I, S, weights, bias):
    X = X.astype(jnp.int32)
    S = S.astype(jnp.int32)
    I = I.astype(jnp.float32)
    xt, ipos = _tc_index(X, I, S)
    return _sc_gather(xt.reshape(-1), ipos, weights, bias)
